# baseline (speedup 1.0000x reference)
import jax
import jax.numpy as jnp
from jax.experimental import pallas as pl
from jax.experimental.pallas import tpu as pltpu

_BBLK = 256
_GW = 16


def _fused_body(x_ref, w0_ref, w1_ref, wh_ref, b_ref, o_ref, *,
                Bblk, H, W, Cin, Cmid):
    pw = (_GW + 1) * Cin
    xb = x_ref[...]
    zrow = jnp.zeros((Bblk, 1, W * Cin), xb.dtype)
    xm = jnp.concatenate([zrow, xb[:, : H - 1]], axis=1)
    xp = jnp.concatenate([xb[:, 1:], zrow], axis=1)

    lo1 = W * Cin - pw
    p0 = jnp.concatenate(
        [xm[:, :, :pw], xb[:, :, :pw], xp[:, :, :pw]], axis=-1)
    p1 = jnp.concatenate(
        [xm[:, :, lo1:], xb[:, :, lo1:], xp[:, :, lo1:]], axis=-1)
    p0 = p0.reshape(Bblk * H, 3 * pw)
    p1 = p1.reshape(Bblk * H, 3 * pw)

    y0 = jnp.dot(p0, w0_ref[...], preferred_element_type=jnp.float32)
    y1 = jnp.dot(p1, w1_ref[...], preferred_element_type=jnp.float32)
    cb = b_ref[0:1, :]
    act = (jnp.maximum(y0 + cb, 0.0) + jnp.maximum(y1 + cb, 0.0))
    s = act.reshape(Bblk, H, _GW * Cmid).sum(axis=1)
    fb = b_ref[1:2, 0:128]
    logits = jnp.dot(s, wh_ref[...],
                     preferred_element_type=jnp.float32) + fb
    o_ref[...] = logits[:, : o_ref.shape[-1]]


def _build_consts(conv_w, conv_b, fc_w, fc_b, H, W):
    KH, KW, Cin, Cmid = conv_w.shape
    n_cls = fc_w.shape[-1]
    pwin = _GW + 1
    conv_w = conv_w.astype(jnp.float32)

    wp_idx = jnp.arange(pwin)
    wo_idx = jnp.arange(_GW)
    dw_idx = jnp.arange(KW)
    sel0 = (wp_idx[:, None, None] == wo_idx[None, :, None] + dw_idx[None, None, :] - 1)
    sel1 = (wp_idx[:, None, None] == wo_idx[None, :, None] + dw_idx[None, None, :])
    w0 = jnp.einsum("pvd,hdic->hpivc", sel0.astype(jnp.float32), conv_w)
    w1 = jnp.einsum("pvd,hdic->hpivc", sel1.astype(jnp.float32), conv_w)
    w0 = w0.reshape(KH * pwin * Cin, _GW * Cmid).astype(jnp.bfloat16)
    w1 = w1.reshape(KH * pwin * Cin, _GW * Cmid).astype(jnp.bfloat16)

    ssum = jnp.tile(jnp.eye(Cmid, dtype=jnp.float32), (_GW, 1))
    whead = (ssum @ fc_w[:Cmid].astype(jnp.float32)) * (1.0 / (H * W))
    whead = jnp.pad(whead, ((0, 0), (0, 128 - n_cls)))

    bias2 = jnp.zeros((8, _GW * Cmid), jnp.float32)
    bias2 = bias2.at[0, :].set(jnp.tile(conv_b.astype(jnp.float32), _GW))
    bias2 = bias2.at[1, :n_cls].set(fc_b.astype(jnp.float32))
    return w0, w1, whead, bias2


def kernel(conv_w, conv_b, fc_w, fc_b, paired_img):
    B, Cin, H, W = paired_img.shape
    KH, KW, _, Cmid = conv_w.shape
    n_cls = fc_w.shape[-1]
    Bblk = min(_BBLK, B)
    nB = pl.cdiv(B, Bblk)
    B_pad = nB * Bblk

    w0, w1, whead, bias2 = _build_consts(conv_w, conv_b, fc_w, fc_b, H, W)

    x16 = jax.lax.optimization_barrier(paired_img.astype(jnp.bfloat16))
    x = jnp.transpose(x16, (0, 2, 3, 1))
    x = x.reshape(B, H, W * Cin)
    if B_pad != B:
        x = jnp.pad(x, ((0, B_pad - B), (0, 0), (0, 0)))

    body = lambda *refs: _fused_body(*refs, Bblk=Bblk, H=H, W=W,
                                     Cin=Cin, Cmid=Cmid)
    out = pl.pallas_call(
        body,
        out_shape=jax.ShapeDtypeStruct((B_pad, n_cls), jnp.float32),
        grid_spec=pltpu.PrefetchScalarGridSpec(
            num_scalar_prefetch=0,
            grid=(nB,),
            in_specs=[
                pl.BlockSpec((Bblk, H, W * Cin), lambda b: (b, 0, 0)),
                pl.BlockSpec(w0.shape, lambda b: (0, 0)),
                pl.BlockSpec(w1.shape, lambda b: (0, 0)),
                pl.BlockSpec(whead.shape, lambda b: (0, 0)),
                pl.BlockSpec(bias2.shape, lambda b: (0, 0)),
            ],
            out_specs=pl.BlockSpec((Bblk, n_cls), lambda b: (b, 0)),
        ),
        compiler_params=pltpu.CompilerParams(
            dimension_semantics=("parallel",)),
    )(x, w0, w1, whead, bias2)
    return out[:B]


# speedup vs baseline: 1.0756x; 1.0756x over previous
import jax
import jax.numpy as jnp
from jax.experimental import pallas as pl
from jax.experimental.pallas import tpu as pltpu

_BBLK = 256
_GW = 16


def _fused_body(x_ref, w0_ref, w1_ref, wh_ref, b_ref, o_ref, *,
                Bblk, H, W, Cin, Cmid):
    pw = (_GW + 1) * Cin
    xb = x_ref[...]
    zrow = jnp.zeros((Bblk, 1, W * Cin), xb.dtype)
    xm = jnp.concatenate([zrow, xb[:, : H - 1]], axis=1)
    xp = jnp.concatenate([xb[:, 1:], zrow], axis=1)

    lo1 = W * Cin - pw
    ones = jnp.ones((Bblk, H, 1), xb.dtype)
    p0 = jnp.concatenate(
        [xm[:, :, :pw], xb[:, :, :pw], xp[:, :, :pw], ones], axis=-1)
    p1 = jnp.concatenate(
        [xm[:, :, lo1:], xb[:, :, lo1:], xp[:, :, lo1:], ones], axis=-1)
    p0 = p0.reshape(Bblk * H, 3 * pw + 1)
    p1 = p1.reshape(Bblk * H, 3 * pw + 1)

    y0 = jnp.dot(p0, w0_ref[...], preferred_element_type=jnp.float32)
    y1 = jnp.dot(p1, w1_ref[...], preferred_element_type=jnp.float32)
    act = jnp.maximum(y0, 0.0) + jnp.maximum(y1, 0.0)
    s = act.reshape(Bblk, H, _GW * Cmid).sum(axis=1)
    fb = b_ref[1:2, 0:128]
    logits = jnp.dot(s, wh_ref[...],
                     preferred_element_type=jnp.float32) + fb
    o_ref[...] = logits[:, : o_ref.shape[-1]]


def _build_consts(conv_w, conv_b, fc_w, fc_b, H, W):
    KH, KW, Cin, Cmid = conv_w.shape
    n_cls = fc_w.shape[-1]
    pwin = _GW + 1
    conv_w = conv_w.astype(jnp.float32)

    wp_idx = jnp.arange(pwin)
    wo_idx = jnp.arange(_GW)
    dw_idx = jnp.arange(KW)
    sel0 = (wp_idx[:, None, None] == wo_idx[None, :, None] + dw_idx[None, None, :] - 1)
    sel1 = (wp_idx[:, None, None] == wo_idx[None, :, None] + dw_idx[None, None, :])
    w0 = jnp.einsum("pvd,hdic->hpivc", sel0.astype(jnp.float32), conv_w)
    w1 = jnp.einsum("pvd,hdic->hpivc", sel1.astype(jnp.float32), conv_w)
    cbrow = jnp.tile(conv_b.astype(jnp.float32), _GW)[None, :]
    w0 = jnp.concatenate([w0.reshape(KH * pwin * Cin, _GW * Cmid), cbrow],
                         axis=0).astype(jnp.bfloat16)
    w1 = jnp.concatenate([w1.reshape(KH * pwin * Cin, _GW * Cmid), cbrow],
                         axis=0).astype(jnp.bfloat16)

    ssum = jnp.tile(jnp.eye(Cmid, dtype=jnp.float32), (_GW, 1))
    whead = (ssum @ fc_w[:Cmid].astype(jnp.float32)) * (1.0 / (H * W))
    whead = jnp.pad(whead, ((0, 0), (0, 128 - n_cls)))

    bias2 = jnp.zeros((8, _GW * Cmid), jnp.float32)
    bias2 = bias2.at[0, :].set(jnp.tile(conv_b.astype(jnp.float32), _GW))
    bias2 = bias2.at[1, :n_cls].set(fc_b.astype(jnp.float32))
    return w0, w1, whead, bias2


def kernel(conv_w, conv_b, fc_w, fc_b, paired_img):
    B, Cin, H, W = paired_img.shape
    KH, KW, _, Cmid = conv_w.shape
    n_cls = fc_w.shape[-1]
    Bblk = min(_BBLK, B)
    nB = pl.cdiv(B, Bblk)
    B_pad = nB * Bblk

    w0, w1, whead, bias2 = _build_consts(conv_w, conv_b, fc_w, fc_b, H, W)

    x16 = jax.lax.optimization_barrier(paired_img.astype(jnp.bfloat16))
    x = jnp.transpose(x16, (0, 2, 3, 1))
    x = x.reshape(B, H, W * Cin)
    if B_pad != B:
        x = jnp.pad(x, ((0, B_pad - B), (0, 0), (0, 0)))

    body = lambda *refs: _fused_body(*refs, Bblk=Bblk, H=H, W=W,
                                     Cin=Cin, Cmid=Cmid)
    out = pl.pallas_call(
        body,
        out_shape=jax.ShapeDtypeStruct((B_pad, n_cls), jnp.float32),
        grid_spec=pltpu.PrefetchScalarGridSpec(
            num_scalar_prefetch=0,
            grid=(nB,),
            in_specs=[
                pl.BlockSpec((Bblk, H, W * Cin), lambda b: (b, 0, 0)),
                pl.BlockSpec(w0.shape, lambda b: (0, 0)),
                pl.BlockSpec(w1.shape, lambda b: (0, 0)),
                pl.BlockSpec(whead.shape, lambda b: (0, 0)),
                pl.BlockSpec(bias2.shape, lambda b: (0, 0)),
            ],
            out_specs=pl.BlockSpec((Bblk, n_cls), lambda b: (b, 0)),
        ),
        compiler_params=pltpu.CompilerParams(
            dimension_semantics=("parallel",)),
    )(x, w0, w1, whead, bias2)
    return out[:B]
